# baseline (speedup 1.0000x reference)
import functools

import jax
import jax.numpy as jnp
from jax import lax
from jax.experimental import pallas as pl
from jax.experimental.pallas import tpu as pltpu
from jax.experimental.pallas import tpu_sc as plsc

H = 1024
HC = H // 16
NW = 32
EPS = 1e-5
MAXPOS = 2048
T = 64


def _rsqrt(v):
    vi = lax.bitcast_convert_type(v, jnp.int32)
    y = lax.bitcast_convert_type(jnp.int32(0x5F3759DF) - (vi >> 1), jnp.float32)
    for _ in range(3):
        y = y * (1.5 - 0.5 * v * y * y)
    return y


def _sc_embed_ln(ids, tts, weight, tt_emb, pos_emb, gamma, beta):
    tok = ids.shape[0]
    tpw = tok // NW
    ng = tpw // T
    mesh = plsc.VectorSubcoreMesh(core_axis_name="c", subcore_axis_name="s")

    @functools.partial(
        pl.kernel,
        out_type=jax.ShapeDtypeStruct((tok, H), jnp.float32),
        mesh=mesh,
        scratch_types=[
            pltpu.VMEM((tpw,), jnp.int32),
            pltpu.VMEM((tpw,), jnp.int32),
            pltpu.VMEM((4, H), jnp.float32),
            pltpu.VMEM((2, H), jnp.float32),
            pltpu.VMEM((H,), jnp.float32),
            pltpu.VMEM((H,), jnp.float32),
            pltpu.VMEM((T, H), jnp.float32),
            pltpu.SemaphoreType.DMA,
        ],
        compiler_params=pltpu.CompilerParams(needs_layout_passes=False),
    )
    def k(ids_hbm, tts_hbm, w_hbm, tte_hbm, pos_hbm, gam_hbm, bet_hbm,
          out_hbm, idx_v, tt_v, cmb_v, tte_v, gam_v, bet_v, rows_v, sem):
        wid = lax.axis_index("s") * 2 + lax.axis_index("c")
        base = wid * tpw
        pltpu.sync_copy(ids_hbm.at[pl.ds(base, tpw)], idx_v)
        pltpu.sync_copy(tts_hbm.at[pl.ds(base, tpw)], tt_v)
        pltpu.sync_copy(pos_hbm.at[0], cmb_v.at[0])
        pltpu.sync_copy(pos_hbm.at[0], cmb_v.at[1])
        pltpu.sync_copy(pos_hbm.at[MAXPOS - 1], cmb_v.at[2])
        pltpu.sync_copy(pos_hbm.at[MAXPOS - 1], cmb_v.at[3])
        pltpu.sync_copy(tte_hbm, tte_v)
        pltpu.sync_copy(gam_hbm, gam_v)
        pltpu.sync_copy(bet_hbm, bet_v)

        def addtt(kk, _):
            sl = pl.ds(kk * 16, 16)
            for c in range(4):
                cmb_v[c, sl] = cmb_v[c, sl] + tte_v[c & 1, sl]
            return 0

        lax.fori_loop(0, HC, addtt, 0)

        for g in range(ng):
            pltpu.async_copy(w_hbm.at[idx_v.at[pl.ds(g * T, T)]], rows_v,
                             sem).wait()

            def group_body(tg, _):
                idv = idx_v[pl.ds(g * T + tg * 16, 16)]
                ttv = tt_v[pl.ds(g * T + tg * 16, 16)]
                cvec = jnp.where(idv != 0, 2, 0) + ttv

                for j in range(16):
                    t = tg * 16 + j
                    c = cvec[j]

                    def p1(kk, carry):
                        s, q = carry
                        sl = pl.ds(kk * 16, 16)
                        x = rows_v[t, sl] + cmb_v[c, sl]
                        rows_v[t, sl] = x
                        return (s + x, q + x * x)

                    z = jnp.zeros((16,), jnp.float32)
                    s, q = lax.fori_loop(0, HC, p1, (z, z))
                    mean = jnp.sum(s) * (1.0 / H)
                    var = jnp.sum(q) * (1.0 / H) - mean * mean
                    r = _rsqrt(var + EPS)

                    def p2(kk, _):
                        sl = pl.ds(kk * 16, 16)
                        x = rows_v[t, sl]
                        rows_v[t, sl] = ((x - mean) * r) * gam_v[sl] + bet_v[sl]
                        return 0

                    lax.fori_loop(0, HC, p2, 0)
                return 0

            lax.fori_loop(0, T // 16, group_body, 0)
            pltpu.sync_copy(rows_v, out_hbm.at[pl.ds(base + g * T, T)])

    return k(ids, tts, weight, tt_emb, pos_emb, gamma, beta)


def kernel(input_ids, token_type_ids, weight, token_type_embeddings,
           position_embeddings, gamma, beta):
    b, s = input_ids.shape
    ids = input_ids.reshape(-1)
    tts = token_type_ids.reshape(-1)
    out = _sc_embed_ln(ids, tts, weight, token_type_embeddings,
                       position_embeddings, gamma, beta)
    return out.reshape(b, s, H)
